# baseline (speedup 1.0000x reference)
import functools

import jax
import jax.numpy as jnp
from jax.experimental import pallas as pl
from jax.experimental.pallas import tpu as pltpu

_C1, _C2, _C3 = 32, 64, 64
_FC1, _APAD = 512, 128
_ACT = 18
_BF = jnp.bfloat16
_F32 = jnp.float32


def _r8(n):
    return ((n + 7) // 8) * 8


def _conv_body(x3_ref, w1_ref, b1_ref, w2_ref, b2_ref, w3_ref, b3_ref,
               o3_ref, xp, o1, sd, o2, *, dims):
    Bt, OH1, OW1, OH2, OW2, OH3, OW3, M2, SDR, M3 = dims
    HW1 = OW1 // 2
    SW = HW1
    SH = OH1 // 2
    H4 = OH1 + 1
    P1 = 16
    M1 = _r8((OH1 - 1) * P1 + HW1)
    KW1 = 2 * x3_ref.shape[-1]

    U = x3_ref.shape[2] // 2
    ev0 = x3_ref[:, :, 0:HW1, :]
    odd = x3_ref[:, :, U:U + HW1, :]
    ev1 = x3_ref[:, :, 1:1 + HW1, :]
    half = KW1 // 2
    xp[:, 0, :, 0:HW1, 0:half] = ev0
    xp[:, 0, :, 0:HW1, half:KW1] = odd
    xp[:, 1, :, 0:HW1, 0:half] = odd
    xp[:, 1, :, 0:HW1, half:KW1] = ev1

    for pw in range(2):
        slab = xp[:, pw, :, :, :].reshape(Bt, H4 * P1, KW1)
        y = jnp.dot(slab[:, 0:M1, :].reshape(Bt * M1, KW1),
                    w1_ref[0:KW1, :], preferred_element_type=_F32)
        y += jnp.dot(slab[:, P1:P1 + M1, :].reshape(Bt * M1, KW1),
                     w1_ref[KW1:2 * KW1, :], preferred_element_type=_F32)
        y = jnp.maximum(y + b1_ref[...], 0.0).astype(_BF)
        o1[:, :, pw * _C1:(pw + 1) * _C1] = y.reshape(Bt, M1, _C1)

    for i in range(SH):
        for ph in range(2):
            sd[:, i * SW:(i + 1) * SW, ph * 2 * _C1:(ph + 1) * 2 * _C1] = (
                o1[:, (2 * i + ph) * P1:(2 * i + ph) * P1 + SW, :])
    sd[:, SH * SW:SDR, :] = jnp.zeros((Bt, SDR - SH * SW, 4 * _C1), _BF)

    y2 = None
    for kh in range(2):
        for kw in range(2):
            off = kh * SW + kw
            t = jnp.dot(sd[:, off:off + M2, :].reshape(Bt * M2, 4 * _C1),
                        w2_ref[(kh * 2 + kw) * 4 * _C1:
                               (kh * 2 + kw + 1) * 4 * _C1, :],
                        preferred_element_type=_F32)
            y2 = t if y2 is None else y2 + t
    y2 = jnp.maximum(y2 + b2_ref[...], 0.0).astype(_BF)
    o2[...] = y2.reshape(Bt, M2, _C2)

    y3 = None
    for kh in range(3):
        for kw in range(3):
            off = kh * SW + kw
            t = jnp.dot(o2[:, off:off + M3, :].reshape(Bt * M3, _C2),
                        w3_ref[(kh * 3 + kw) * _C2:(kh * 3 + kw + 1) * _C2, :],
                        preferred_element_type=_F32)
            y3 = t if y3 is None else y3 + t
    y3 = jnp.maximum(y3 + b3_ref[...], 0.0).astype(_BF)
    o3_ref[...] = y3.reshape(Bt, M3, _C3)


def _fc_body(x_ref, wf1_ref, bf1_ref, wf2_ref, bf2_ref, q_ref):
    Bm, M3, C = x_ref.shape
    x = x_ref[...].reshape(Bm, M3 * C)
    h = jnp.dot(x, wf1_ref[...], preferred_element_type=_F32)
    h = jnp.maximum(h + bf1_ref[...], 0.0).astype(_BF)
    q_ref[...] = (jnp.dot(h, wf2_ref[...], preferred_element_type=_F32)
                  + bf2_ref[...])


def _prep(state):
    B, C, H, W = state.shape
    H4, W4 = H // 4, W // 4
    U = (W4 + 1) // 2
    x = state.astype(_BF)[:, :, :H4 * 4, :]
    x = jnp.pad(x, ((0, 0), (0, 0), (0, 0), (0, 4 * 2 * U - W)))
    x = x.reshape(B, C, H4, 4, U, 2, 4)
    x = x.transpose(0, 2, 5, 4, 3, 6, 1)
    return x.reshape(B, H4, 2 * U, 16 * C)


def _wspec(a):
    nd = a.ndim
    return pl.BlockSpec(a.shape, lambda s, _n=nd: (0,) * _n)


@jax.jit
def _forward(w1, b1, w2, b2, w3, b3, wf1, bf1, wf2, bf2, state):
    B, C, H, W = state.shape
    H4, W4 = H // 4, W // 4
    OH1, OW1 = H4 - 1, W4 - 1
    OH2, OW2 = OH1 // 2 - 1, OW1 // 2 - 1
    OH3, OW3 = OH2 - 2, OW2 - 2
    assert OH1 % 2 == 0 and OW1 % 2 == 0 and OH3 >= 1 and OW3 >= 1
    HW1 = OW1 // 2
    SW = HW1
    RH = OH1 * HW1
    assert RH % 8 == 0
    M3 = _r8((OH3 - 1) * SW + OW3)
    M2 = _r8(max((OH2 - 1) * SW + OW2, 2 * SW + 2 + M3))
    SDR = _r8(M2 + SW + 1)

    P1 = 16
    M1 = _r8((OH1 - 1) * P1 + HW1)

    Bt = 16 if (B >= 32 and B % 16 == 0) else max(1, min(8, B))
    Bpad = -(-B // Bt) * Bt

    x3 = _prep(state)
    if Bpad != B:
        x3 = jnp.pad(x3, ((0, Bpad - B), (0, 0), (0, 0), (0, 0)))

    w1b, w2b, w3b = w1.astype(_BF), w2.astype(_BF), w3.astype(_BF)

    in_block = (Bt,) + x3.shape[1:]
    body = functools.partial(
        _conv_body, dims=(Bt, OH1, OW1, OH2, OW2, OH3, OW3, M2, SDR, M3))
    o3 = pl.pallas_call(
        body,
        out_shape=jax.ShapeDtypeStruct((Bpad, M3, _C3), _BF),
        grid=(Bpad // Bt,),
        in_specs=[
            pl.BlockSpec(in_block, lambda s: (s, 0, 0, 0)),
            _wspec(w1b), _wspec(b1), _wspec(w2b), _wspec(b2),
            _wspec(w3b), _wspec(b3),
        ],
        out_specs=pl.BlockSpec((Bt, M3, _C3), lambda s: (s, 0, 0)),
        scratch_shapes=[
            pltpu.VMEM((Bt, 2, H4, P1, 32 * C), _BF),
            pltpu.VMEM((Bt, M1, 2 * _C1), _BF),
            pltpu.VMEM((Bt, SDR, 4 * _C1), _BF),
            pltpu.VMEM((Bt, M2, _C2), _BF),
        ],
        compiler_params=pltpu.CompilerParams(
            dimension_semantics=("parallel",),
            vmem_limit_bytes=64 * 1024 * 1024),
    )(x3, w1b, b1, w2b, b2, w3b, b3)

    wf1q = wf1.reshape(OH3, OW3, _C3, _FC1)
    wf1q = jnp.pad(wf1q, ((0, 0), (0, SW - OW3), (0, 0), (0, 0)))
    wf1q = wf1q.reshape(OH3 * SW * _C3, _FC1)
    wf1q = jnp.pad(wf1q, ((0, (M3 - OH3 * SW) * _C3), (0, 0))).astype(_BF)
    wf2b = wf2.astype(_BF)

    Bm = Bpad if Bpad <= 128 else 128
    Bf = -(-Bpad // Bm) * Bm
    if Bf != Bpad:
        o3 = jnp.pad(o3, ((0, Bf - Bpad), (0, 0), (0, 0)))

    q = pl.pallas_call(
        _fc_body,
        out_shape=jax.ShapeDtypeStruct((Bf, _APAD), _F32),
        grid=(Bf // Bm,),
        in_specs=[
            pl.BlockSpec((Bm, M3, _C3), lambda s: (s, 0, 0)),
            _wspec(wf1q), _wspec(bf1), _wspec(wf2b), _wspec(bf2),
        ],
        out_specs=pl.BlockSpec((Bm, _APAD), lambda s: (s, 0)),
        compiler_params=pltpu.CompilerParams(
            dimension_semantics=("parallel",),
            vmem_limit_bytes=64 * 1024 * 1024),
    )(o3, wf1q, bf1, wf2b, bf2)
    return q[:B, :_ACT]


def kernel(w1, b1, w2, b2, w3, b3, wf1, bf1, wf2, bf2, state):
    return _forward(w1, b1, w2, b2, w3, b3, wf1, bf1, wf2, bf2, state)


# speedup vs baseline: 1.1578x; 1.1578x over previous
import functools

import jax
import jax.numpy as jnp
from jax.experimental import pallas as pl
from jax.experimental.pallas import tpu as pltpu

_C1, _C2, _C3 = 32, 64, 64
_FC1, _APAD = 512, 128
_ACT = 18
_BF = jnp.bfloat16
_F32 = jnp.float32


def _r8(n):
    return ((n + 7) // 8) * 8


def _conv_body(xe_ref, xo_ref, w1_ref, b1_ref, w2_ref, b2_ref, w3_ref, b3_ref,
               o3_ref, o1, sd, o2, *, dims):
    Bt, OH1, OW1, OH2, OW2, OH3, OW3, M2, SDR, M3 = dims
    HW1 = OW1 // 2
    SW = HW1
    SH = OH1 // 2
    RH = OH1 * HW1
    KW1 = xe_ref.shape[-1]

    for pw, src in enumerate((xe_ref, xo_ref)):
        y = jnp.dot(src[:, 0:RH, :].reshape(Bt * RH, KW1),
                    w1_ref[0:KW1, :], preferred_element_type=_F32)
        y += jnp.dot(src[:, HW1:HW1 + RH, :].reshape(Bt * RH, KW1),
                     w1_ref[KW1:2 * KW1, :], preferred_element_type=_F32)
        y = jnp.maximum(y + b1_ref[...], 0.0).astype(_BF)
        o1[:, :, pw * _C1:(pw + 1) * _C1] = y.reshape(Bt, RH, _C1)

    for i in range(SH):
        for ph in range(2):
            sd[:, i * SW:(i + 1) * SW, ph * 2 * _C1:(ph + 1) * 2 * _C1] = (
                o1[:, (2 * i + ph) * SW:(2 * i + ph + 1) * SW, :])
    sd[:, SH * SW:SDR, :] = jnp.zeros((Bt, SDR - SH * SW, 4 * _C1), _BF)

    y2 = None
    for kh in range(2):
        for kw in range(2):
            off = kh * SW + kw
            t = jnp.dot(sd[:, off:off + M2, :].reshape(Bt * M2, 4 * _C1),
                        w2_ref[(kh * 2 + kw) * 4 * _C1:
                               (kh * 2 + kw + 1) * 4 * _C1, :],
                        preferred_element_type=_F32)
            y2 = t if y2 is None else y2 + t
    y2 = jnp.maximum(y2 + b2_ref[...], 0.0).astype(_BF)
    o2[...] = y2.reshape(Bt, M2, _C2)

    y3 = None
    for kh in range(3):
        for kw in range(3):
            off = kh * SW + kw
            t = jnp.dot(o2[:, off:off + M3, :].reshape(Bt * M3, _C2),
                        w3_ref[(kh * 3 + kw) * _C2:(kh * 3 + kw + 1) * _C2, :],
                        preferred_element_type=_F32)
            y3 = t if y3 is None else y3 + t
    y3 = jnp.maximum(y3 + b3_ref[...], 0.0).astype(_BF)
    o3_ref[...] = y3.reshape(Bt, M3, _C3)


def _fc_body(x_ref, wf1_ref, bf1_ref, wf2_ref, bf2_ref, q_ref, *, fdims):
    OH3, OW3, SW = fdims
    Bm, M3, C = x_ref.shape
    x = x_ref[...].reshape(Bm, M3 * C)
    h = None
    for g in range(OH3):
        t = jnp.dot(x[:, g * SW * C:g * SW * C + OW3 * C],
                    wf1_ref[g * OW3 * C:(g + 1) * OW3 * C, :],
                    preferred_element_type=_F32)
        h = t if h is None else h + t
    h = jnp.maximum(h + bf1_ref[...], 0.0).astype(_BF)
    q_ref[...] = (jnp.dot(h, wf2_ref[...], preferred_element_type=_F32)
                  + bf2_ref[...])


def _prep(state):
    B, C, H, W = state.shape
    H4, W4 = H // 4, W // 4
    x = state.astype(jnp.uint8).transpose(0, 2, 3, 1)[:, :H4 * 4, :W4 * 4, :]
    x = x.reshape(B, H4, 4, W4, 4, C).transpose(0, 1, 3, 2, 4, 5)
    x = x.reshape(B, H4, W4, 16 * C)
    HW1 = (W4 - 1) // 2
    xe = x[:, :, 0:2 * HW1, :].reshape(B, H4 * HW1, 32 * C).astype(_BF)
    xo = x[:, :, 1:1 + 2 * HW1, :].reshape(B, H4 * HW1, 32 * C).astype(_BF)
    return xe, xo


def _wspec(a):
    nd = a.ndim
    return pl.BlockSpec(a.shape, lambda s, _n=nd: (0,) * _n)


@jax.jit
def _forward(w1, b1, w2, b2, w3, b3, wf1, bf1, wf2, bf2, state):
    B, C, H, W = state.shape
    H4, W4 = H // 4, W // 4
    OH1, OW1 = H4 - 1, W4 - 1
    OH2, OW2 = OH1 // 2 - 1, OW1 // 2 - 1
    OH3, OW3 = OH2 - 2, OW2 - 2
    assert OH1 % 2 == 0 and OW1 % 2 == 0 and OH3 >= 1 and OW3 >= 1
    HW1 = OW1 // 2
    SW = HW1
    RH = OH1 * HW1
    assert RH % 8 == 0
    M3 = _r8((OH3 - 1) * SW + OW3)
    M2 = _r8(max((OH2 - 1) * SW + OW2, 2 * SW + 2 + M3))
    SDR = _r8(M2 + SW + 1)

    Bt = 16 if (B >= 32 and B % 16 == 0) else max(1, min(8, B))
    Bpad = -(-B // Bt) * Bt

    xe, xo = _prep(state)
    if Bpad != B:
        pad = ((0, Bpad - B), (0, 0), (0, 0))
        xe = jnp.pad(xe, pad)
        xo = jnp.pad(xo, pad)

    w1b, w2b, w3b = w1.astype(_BF), w2.astype(_BF), w3.astype(_BF)

    in_block = (Bt,) + xe.shape[1:]
    body = functools.partial(
        _conv_body, dims=(Bt, OH1, OW1, OH2, OW2, OH3, OW3, M2, SDR, M3))
    o3 = pl.pallas_call(
        body,
        out_shape=jax.ShapeDtypeStruct((Bpad, M3, _C3), _BF),
        grid=(Bpad // Bt,),
        in_specs=[
            pl.BlockSpec(in_block, lambda s: (s, 0, 0)),
            pl.BlockSpec(in_block, lambda s: (s, 0, 0)),
            _wspec(w1b), _wspec(b1), _wspec(w2b), _wspec(b2),
            _wspec(w3b), _wspec(b3),
        ],
        out_specs=pl.BlockSpec((Bt, M3, _C3), lambda s: (s, 0, 0)),
        scratch_shapes=[
            pltpu.VMEM((Bt, RH, 2 * _C1), _BF),
            pltpu.VMEM((Bt, SDR, 4 * _C1), _BF),
            pltpu.VMEM((Bt, M2, _C2), _BF),
        ],
        compiler_params=pltpu.CompilerParams(
            dimension_semantics=("parallel",),
            vmem_limit_bytes=64 * 1024 * 1024),
    )(xe, xo, w1b, b1, w2b, b2, w3b, b3)

    wf1b = wf1.astype(_BF)
    wf2b = wf2.astype(_BF)

    Bm = Bpad if Bpad <= 128 else 128
    Bf = -(-Bpad // Bm) * Bm
    if Bf != Bpad:
        o3 = jnp.pad(o3, ((0, Bf - Bpad), (0, 0), (0, 0)))

    fbody = functools.partial(_fc_body, fdims=(OH3, OW3, SW))
    q = pl.pallas_call(
        fbody,
        out_shape=jax.ShapeDtypeStruct((Bf, _APAD), _F32),
        grid=(Bf // Bm,),
        in_specs=[
            pl.BlockSpec((Bm, M3, _C3), lambda s: (s, 0, 0)),
            _wspec(wf1b), _wspec(bf1), _wspec(wf2b), _wspec(bf2),
        ],
        out_specs=pl.BlockSpec((Bm, _APAD), lambda s: (s, 0)),
        compiler_params=pltpu.CompilerParams(
            dimension_semantics=("parallel",),
            vmem_limit_bytes=64 * 1024 * 1024),
    )(o3, wf1b, bf1, wf2b, bf2)
    return q[:B, :_ACT]


def kernel(w1, b1, w2, b2, w3, b3, wf1, bf1, wf2, bf2, state):
    return _forward(w1, b1, w2, b2, w3, b3, wf1, bf1, wf2, bf2, state)
